# baseline (speedup 1.0000x reference)
import dataclasses as _dc

import jax
import jax.numpy as jnp
from jax import lax
from jax.experimental import pallas as pl
from jax.experimental.pallas import tpu as pltpu
from jax.experimental.pallas import tpu_sc as plsc

N = 240000
NX, NY, B = 352, 400, 2
NUM_SEG = B * NX * NY
VX = 0.2
VY = 0.2
XOFF = VX / 2 + 0.0
YOFF = VY / 2 + (-40.0)
EPS = 1e-3
C_OUT = 64

NW = 32
WIN = 1600
NWIN = N // WIN
WPT = (NWIN + NW - 1) // NW
GRP = WIN // 16

SPB = 1024
NBINS = 288
NBINS_REAL = NUM_SEG // SPB
NPASS = 9
NEG = -3.0e38

N2 = 128000
CORE_N = (128000, 112000)
CORE_NWIN = (80, 70)

_mesh = plsc.VectorSubcoreMesh(core_axis_name="c", subcore_axis_name="s")
_sc_params = pltpu.CompilerParams(
    needs_layout_passes=False, use_tc_tiling_on_sc=False)

_I16 = lambda: lax.iota(jnp.int32, 16)


def _wid():
    return lax.axis_index("c") * 16 + lax.axis_index("s")


def _sget(vec, lane):
    return jnp.sum(jnp.where(_I16() == lane, vec, 0))


def _stage_a_body(coors_hbm, feats_hbm, seg_hbm, tables_hbm, hist_hbm,
                  cbuf, fbuf, segbuf, ix0, ix1, ix2, ix3, vx, vy, vz,
                  ones_b, zbuf, hcnt, tbl_sh):
    cid = lax.axis_index("c")
    sid = lax.axis_index("s")
    wid = cid * 16 + sid

    @pl.loop(0, 8800, step=16)
    def _(i):
        zbuf[pl.ds(i, 16)] = jnp.zeros((16,), jnp.float32)

    @pl.loop(0, 8, step=1)
    def _(j):
        pltpu.sync_copy(zbuf, tbl_sh.at[pl.ds(sid * 70400 + j * 8800, 8800)])

    @pl.loop(0, WIN, step=16)
    def _(i):
        ones_b[pl.ds(i, 16)] = jnp.ones((16,), jnp.float32)

    @pl.loop(0, NBINS, step=16)
    def _(i):
        hcnt[pl.ds(i, 16)] = jnp.zeros((16,), jnp.int32)

    occ_p, _ = plsc.scan_count(_I16())
    occ_base = jnp.min(occ_p)

    plsc.subcore_barrier()

    @pl.loop(0, WPT)
    def _(k):
        widx = wid + k * NW

        @pl.when(widx < NWIN)
        def _():
            base = widx * WIN
            pltpu.sync_copy(coors_hbm.at[pl.ds(base, WIN)], cbuf)
            pltpu.sync_copy(feats_hbm.at[pl.ds(base, WIN)], fbuf)

            @pl.loop(0, GRP)
            def _(g):
                rows = g * 16 + _I16()
                zc = jnp.zeros((16,), jnp.int32)
                b16 = plsc.load_gather(cbuf, [rows, zc])
                y16 = plsc.load_gather(cbuf, [rows, zc + 2])
                x16 = plsc.load_gather(cbuf, [rows, zc + 3])
                s16 = b16 * (NX * NY) + y16 * NX + x16
                segbuf[pl.ds(g * 16, 16)] = s16
                t4 = s16 * 4
                ix0[pl.ds(g * 16, 16)] = t4
                ix1[pl.ds(g * 16, 16)] = t4 + 1
                ix2[pl.ds(g * 16, 16)] = t4 + 2
                ix3[pl.ds(g * 16, 16)] = t4 + 3
                vx[pl.ds(g * 16, 16)] = plsc.load_gather(fbuf, [rows, zc])
                vy[pl.ds(g * 16, 16)] = plsc.load_gather(fbuf, [rows, zc + 1])
                vz[pl.ds(g * 16, 16)] = plsc.load_gather(fbuf, [rows, zc + 2])

                bin16 = s16 >> 10
                occ, lastm = plsc.scan_count(bin16)
                occ0 = occ - occ_base
                cur = plsc.load_gather(hcnt, [bin16])
                plsc.store_scatter(hcnt, [bin16], cur + occ0 + 1, mask=lastm)

            pltpu.sync_copy(vx, tbl_sh.at[ix0], add=True)
            pltpu.sync_copy(vy, tbl_sh.at[ix1], add=True)
            pltpu.sync_copy(vz, tbl_sh.at[ix2], add=True)
            pltpu.sync_copy(ones_b, tbl_sh.at[ix3], add=True)
            pltpu.sync_copy(segbuf, seg_hbm.at[pl.ds(base, WIN)])

    pltpu.sync_copy(hcnt, hist_hbm.at[pl.ds(wid * NBINS, NBINS)])
    plsc.subcore_barrier()
    pltpu.sync_copy(tbl_sh.at[pl.ds(sid * 70400, 70400)],
                    tables_hbm.at[cid, pl.ds(sid * 70400, 70400)])


def _stage_a(coors, features):
    k = pl.kernel(
        _stage_a_body,
        name="stage_a",
        out_type=(
            jax.ShapeDtypeStruct((N,), jnp.int32),
            jax.ShapeDtypeStruct((2, NUM_SEG * 4), jnp.float32),
            jax.ShapeDtypeStruct((NW * NBINS,), jnp.int32),
        ),
        mesh=_mesh,
        compiler_params=_sc_params,
        scratch_types=[
            pltpu.VMEM((WIN, 4), jnp.int32),
            pltpu.VMEM((WIN, 4), jnp.float32),
            pltpu.VMEM((WIN,), jnp.int32),
            pltpu.VMEM((WIN,), jnp.int32),
            pltpu.VMEM((WIN,), jnp.int32),
            pltpu.VMEM((WIN,), jnp.int32),
            pltpu.VMEM((WIN,), jnp.int32),
            pltpu.VMEM((WIN,), jnp.float32),
            pltpu.VMEM((WIN,), jnp.float32),
            pltpu.VMEM((WIN,), jnp.float32),
            pltpu.VMEM((WIN,), jnp.float32),
            pltpu.VMEM((8800,), jnp.float32),
            pltpu.VMEM((NBINS,), jnp.int32),
            pltpu.VMEM_SHARED((NUM_SEG * 4,), jnp.float32),
        ],
    )
    return k(coors, features)


def _stage_abin_body(seg_hbm, hist_hbm, ppid_hbm, pseg_hbm, binfo_hbm,
                     histbuf, csum, starts, counters,
                     segbuf, posbuf, pidbuf, sp_sh):
    cid = lax.axis_index("c")
    sid = lax.axis_index("s")
    wid = cid * 16 + sid

    pltpu.sync_copy(hist_hbm, histbuf)

    occ_p, _ = plsc.scan_count(_I16())
    occ_base = jnp.min(occ_p)

    @pl.loop(0, NBINS, step=16)
    def _(cb):
        acc = jnp.zeros((16,), jnp.int32)

        def _accf(t, a):
            return a + histbuf[pl.ds(t * NBINS + cb, 16)]

        acc = lax.fori_loop(cid * 16, cid * 16 + 16, _accf, acc)
        csum[pl.ds(cb, 16)] = acc

    def _pref(i, carry):
        v = csum[pl.ds(i * 16, 16)]
        cs = plsc.cumsum(v)
        starts[pl.ds(i * 16, 16)] = carry + cs - v
        return carry + jnp.sum(v)

    lax.fori_loop(0, NBINS // 16, _pref, jnp.int32(0))

    @pl.loop(0, NBINS, step=16)
    def _(cb):
        acc = starts[pl.ds(cb, 16)]

        def _acc(t, a):
            return a + histbuf[pl.ds(t * NBINS + cb, 16)]

        acc = lax.fori_loop(cid * 16, wid, _acc, acc)
        counters[pl.ds(cb, 16)] = acc

    @pl.when(sid == 0)
    def _():
        pltpu.sync_copy(starts, binfo_hbm.at[pl.ds(cid * 2 * NBINS, NBINS)])
        pltpu.sync_copy(csum,
                        binfo_hbm.at[pl.ds(cid * 2 * NBINS + NBINS, NBINS)])

    @pl.loop(0, WPT)
    def _(k):
        widx = wid + k * NW

        @pl.when(widx < NWIN)
        def _():
            base = widx * WIN
            pltpu.sync_copy(seg_hbm.at[pl.ds(base, WIN)], segbuf)

            @pl.loop(0, GRP)
            def _(g):
                s16 = segbuf[pl.ds(g * 16, 16)]
                bin16 = s16 >> 10
                occ, lastm = plsc.scan_count(bin16)
                occ0 = occ - occ_base
                cur = plsc.load_gather(counters, [bin16])
                plsc.store_scatter(counters, [bin16], cur + occ0 + 1,
                                   mask=lastm)
                posbuf[pl.ds(g * 16, 16)] = cur + occ0
                pidbuf[pl.ds(g * 16, 16)] = base + g * 16 + _I16()

            pltpu.sync_copy(pidbuf, sp_sh.at[posbuf])

            @pl.loop(0, GRP)
            def _(g):
                posbuf[pl.ds(g * 16, 16)] = posbuf[pl.ds(g * 16, 16)] + N2

            pltpu.sync_copy(segbuf, sp_sh.at[posbuf])

    plsc.subcore_barrier()
    pltpu.sync_copy(sp_sh.at[pl.ds(sid * 8000, 8000)],
                    ppid_hbm.at[cid, pl.ds(sid * 8000, 8000)])
    pltpu.sync_copy(sp_sh.at[pl.ds(N2 + sid * 8000, 8000)],
                    pseg_hbm.at[cid, pl.ds(sid * 8000, 8000)])


def _stage_abin(seg, hist):
    k = pl.kernel(
        _stage_abin_body,
        name="stage_abin",
        out_type=(
            jax.ShapeDtypeStruct((2, N2), jnp.int32),
            jax.ShapeDtypeStruct((2, N2), jnp.int32),
            jax.ShapeDtypeStruct((4 * NBINS,), jnp.int32),
        ),
        mesh=_mesh,
        compiler_params=_sc_params,
        scratch_types=[
            pltpu.VMEM((NW * NBINS,), jnp.int32),
            pltpu.VMEM((NBINS,), jnp.int32),
            pltpu.VMEM((NBINS,), jnp.int32),
            pltpu.VMEM((NBINS,), jnp.int32),
            pltpu.VMEM((WIN,), jnp.int32),
            pltpu.VMEM((WIN,), jnp.int32),
            pltpu.VMEM((WIN,), jnp.int32),
            pltpu.VMEM_SHARED((2 * N2,), jnp.int32),
        ],
    )
    return k(seg, hist)


MCH = 6400
NMCH = NUM_SEG * 4 // MCH


def _stage_amerge_body(t0_hbm, t1_hbm, tm_hbm, b0, b1, ob):
    wid = _wid()

    @pl.loop(0, (NMCH + NW - 1) // NW)
    def _(k):
        ci = wid + k * NW

        @pl.when(ci < NMCH)
        def _():
            base = ci * MCH
            pltpu.sync_copy(t0_hbm.at[pl.ds(base, MCH)], b0)
            pltpu.sync_copy(t1_hbm.at[pl.ds(base, MCH)], b1)

            @pl.loop(0, MCH // 64)
            def _(g):
                rows4 = (g * 16 + _I16()) * 4
                sx = (plsc.load_gather(b0, [rows4])
                      + plsc.load_gather(b1, [rows4]))
                sy = (plsc.load_gather(b0, [rows4 + 1])
                      + plsc.load_gather(b1, [rows4 + 1]))
                sz = (plsc.load_gather(b0, [rows4 + 2])
                      + plsc.load_gather(b1, [rows4 + 2]))
                cnt = (plsc.load_gather(b0, [rows4 + 3])
                       + plsc.load_gather(b1, [rows4 + 3]))
                inv = 1.0 / jnp.maximum(cnt, 1.0)
                plsc.store_scatter(ob, [rows4], sx * inv)
                plsc.store_scatter(ob, [rows4 + 1], sy * inv)
                plsc.store_scatter(ob, [rows4 + 2], sz * inv)
                plsc.store_scatter(ob, [rows4 + 3], cnt)

            pltpu.sync_copy(ob, tm_hbm.at[pl.ds(base, MCH)])


def _stage_amerge(tables):
    k = pl.kernel(
        _stage_amerge_body,
        name="stage_amerge",
        out_type=jax.ShapeDtypeStruct((NUM_SEG * 4,), jnp.float32),
        mesh=_mesh,
        compiler_params=_sc_params,
        scratch_types=[
            pltpu.VMEM((MCH,), jnp.float32),
            pltpu.VMEM((MCH,), jnp.float32),
            pltpu.VMEM((MCH,), jnp.float32),
        ],
    )
    return k(tables[0], tables[1])


def _stage_a2_body(ppid_hbm, pseg_hbm, featf_hbm, tm_hbm, pf_hbm,
                   pidb, segb, jf0, jf1, jf2, jf3, jm0, jm1, jm2,
                   r0, r1, r2, r3, r4, r5, r6, r7, r8):
    wid = _wid()
    out_rows = (r0, r1, r2, r3, r4, r5, r6, r7, r8)

    @pl.loop(0, WPT)
    def _(k):
        widx = wid + k * NW

        @pl.when(widx < NWIN)
        def _():
            base = widx * WIN
            src = (widx >= CORE_NWIN[0]).astype(jnp.int32)
            inbase = base - src * N2
            pltpu.sync_copy(ppid_hbm.at[src, pl.ds(inbase, WIN)], pidb)
            pltpu.sync_copy(pseg_hbm.at[src, pl.ds(inbase, WIN)], segb)

            @pl.loop(0, GRP)
            def _(g):
                sl = pl.ds(g * 16, 16)
                pid = pidb[sl]
                pid = jnp.minimum(jnp.maximum(pid, 0), N - 1)
                s16 = segb[sl]
                s16 = jnp.minimum(jnp.maximum(s16, 0), NUM_SEG - 1)
                pidb[sl] = pid * 4
                segb[sl] = s16 * 4
                xv = jnp.remainder(s16, NX)
                yv = jnp.remainder(s16 // NX, NY)
                r7[sl] = yv.astype(jnp.float32)
                r8[sl] = xv.astype(jnp.float32)

            pltpu.sync_copy(featf_hbm.at[pidb], jf0)
            pltpu.sync_copy(tm_hbm.at[segb], jm0)

            @pl.loop(0, GRP)
            def _(g):
                sl = pl.ds(g * 16, 16)
                pidb[sl] = pidb[sl] + 1
                segb[sl] = segb[sl] + 1

            pltpu.sync_copy(featf_hbm.at[pidb], jf1)
            pltpu.sync_copy(tm_hbm.at[segb], jm1)

            @pl.loop(0, GRP)
            def _(g):
                sl = pl.ds(g * 16, 16)
                pidb[sl] = pidb[sl] + 1
                segb[sl] = segb[sl] + 1

            pltpu.sync_copy(featf_hbm.at[pidb], jf2)
            pltpu.sync_copy(tm_hbm.at[segb], jm2)

            @pl.loop(0, GRP)
            def _(g):
                sl = pl.ds(g * 16, 16)
                pidb[sl] = pidb[sl] + 1

            pltpu.sync_copy(featf_hbm.at[pidb], jf3)

            @pl.loop(0, GRP)
            def _(g):
                sl = pl.ds(g * 16, 16)
                r0[sl] = jf0[sl]
                r1[sl] = jf1[sl]
                r2[sl] = jf2[sl]
                r3[sl] = jf3[sl]
                r4[sl] = jf0[sl] - jm0[sl]
                r5[sl] = jf1[sl] - jm1[sl]
                r6[sl] = jf2[sl] - jm2[sl]
                fy = jf1[sl] - (r7[sl] * VY + YOFF)
                fx = jf0[sl] - (r8[sl] * VX + XOFF)
                r7[sl] = fx
                r8[sl] = fy

            for c in range(9):
                pltpu.sync_copy(out_rows[c], pf_hbm.at[c, pl.ds(base, WIN)])


def _stage_a2(ppid, pseg, featf, tm):
    k = pl.kernel(
        _stage_a2_body,
        name="stage_a2",
        out_type=jax.ShapeDtypeStruct((9, N), jnp.float32),
        mesh=_mesh,
        compiler_params=_sc_params,
        scratch_types=(
            [pltpu.VMEM((WIN,), jnp.int32)] * 2
            + [pltpu.VMEM((WIN,), jnp.float32)] * 16
        ),
    )
    return k(ppid, pseg, featf, tm)


BLK = 3200
NBLK = N // BLK


def _stage_b_kernel(pf_ref, w_ref, h_ref, s_ref):
    i = pl.program_id(0)
    f = pf_ref[...]
    w = w_ref[...]
    h = lax.dot_general(f, w, (((0,), (0,)), ((), ())),
                        preferred_element_type=jnp.float32)
    h_ref[...] = h

    @pl.when(i == 0)
    def _():
        s_ref[...] = jnp.zeros_like(s_ref)

    s_ref[0:1, :] += jnp.sum(h, axis=0, keepdims=True)
    s_ref[1:2, :] += jnp.sum(h * h, axis=0, keepdims=True)


def _stage_b(pf, w9):
    return pl.pallas_call(
        _stage_b_kernel,
        grid=(NBLK,),
        in_specs=[
            pl.BlockSpec((9, BLK), lambda i: (0, i)),
            pl.BlockSpec((9, C_OUT), lambda i: (0, 0)),
        ],
        out_specs=[
            pl.BlockSpec((BLK, C_OUT), lambda i: (i, 0)),
            pl.BlockSpec((2, C_OUT), lambda i: (0, 0)),
        ],
        out_shape=[
            jax.ShapeDtypeStruct((N, C_OUT), jnp.float32),
            jax.ShapeDtypeStruct((2, C_OUT), jnp.float32),
        ],
    )(pf, w9)


CH = 512


def _stage_c_body(ht_hbm, pseg_hbm, binfo_hbm, ss_hbm, out_hbm,
                  infob, ssb, tbl, sbuf, vbuf):
    wid = _wid()

    pltpu.sync_copy(binfo_hbm, infob)
    pltpu.sync_copy(ss_hbm, ssb)

    occ_p, _ = plsc.scan_count(_I16())
    occ_base = jnp.min(occ_p)

    @pl.loop(0, NPASS)
    def _(p):
        b = p * NW + wid

        @pl.when(b < NBINS_REAL)
        def _():
            bq = b >> 4
            br = b - bq * 16
            segbase = b * SPB

            negv = jnp.full((16,), NEG, jnp.float32)

            @pl.loop(0, SPB, step=4)
            def _(i):
                for u in range(4):
                    for j in range(4):
                        tbl[i + u, pl.ds(j * 16, 16)] = negv

            for c in range(2):
                start = _sget(infob[pl.ds(c * 2 * NBINS + bq * 16, 16)], br)
                blen = _sget(
                    infob[pl.ds(c * 2 * NBINS + NBINS + bq * 16, 16)], br)
                astart = pl.multiple_of((start >> 4) << 4, 16)
                total = start + blen - astart
                nch = (total + CH - 1) // CH

                @pl.loop(0, nch)
                def _(ci):
                    pos = pl.multiple_of(astart + ci * CH, 16)
                    pltpu.sync_copy(pseg_hbm.at[c, pl.ds(pos, CH)], sbuf)
                    pltpu.sync_copy(ht_hbm.at[pl.ds(c * N2 + pos, CH), :],
                                    vbuf)
                    ngrp = jnp.minimum(total - ci * CH + 15, CH) >> 4

                    @pl.loop(0, ngrp)
                    def _(g):
                        rows = g * 16 + _I16()
                        gidx = ci * CH + rows
                        valid = ((astart + gidx >= start)
                                 & (astart + gidx < start + blen))
                        s16 = sbuf[pl.ds(g * 16, 16)]
                        soff = s16 - segbase
                        soff = jnp.minimum(jnp.maximum(soff, 0), SPB - 1)
                        occ, _lm = plsc.scan_count(soff, mask=valid)
                        occ0 = jnp.where(valid, occ - occ_base, 0)
                        mo = jnp.max(occ0)
                        zc = jnp.zeros((16,), jnp.int32)

                        @pl.loop(0, mo + 1)
                        def _(r):
                            mr = valid & (occ0 == r)
                            for ch in range(C_OUT):
                                v = plsc.load_gather(vbuf, [rows, zc + ch],
                                                     mask=mr)
                                t = plsc.load_gather(tbl, [soff, zc + ch],
                                                     mask=mr)
                                plsc.store_scatter(tbl, [soff, zc + ch],
                                                   jnp.maximum(t, v),
                                                   mask=mr)

            scs = [ssb[pl.ds(16 * j, 16)] for j in range(4)]
            shs = [ssb[pl.ds(64 + 16 * j, 16)] for j in range(4)]

            @pl.loop(0, SPB, step=4)
            def _(r):
                for u in range(4):
                    for j in range(4):
                        sl = pl.ds(16 * j, 16)
                        tbl[r + u, sl] = jnp.maximum(
                            tbl[r + u, sl] * scs[j] + shs[j], 0.0)

            pltpu.sync_copy(tbl, out_hbm.at[pl.ds(b * SPB, SPB), :])


def _stage_c(ht, pseg, binfo, ss):
    k = pl.kernel(
        _stage_c_body,
        name="stage_c",
        out_type=jax.ShapeDtypeStruct((NUM_SEG, C_OUT), jnp.float32),
        mesh=_mesh,
        compiler_params=_sc_params,
        scratch_types=[
            pltpu.VMEM((4 * NBINS,), jnp.int32),
            pltpu.VMEM((128,), jnp.float32),
            pltpu.VMEM((SPB, C_OUT), jnp.float32),
            pltpu.VMEM((CH,), jnp.int32),
            pltpu.VMEM((CH, C_OUT), jnp.float32),
        ],
    )
    return k(ht, pseg, binfo, ss)


def kernel(features, coors, W0, gamma, beta):
    seg, tables, hist = _stage_a(coors, features)
    ppid, pseg, binfo = _stage_abin(seg, hist)
    tm = _stage_amerge(tables)
    pf = _stage_a2(ppid, pseg, features.reshape(N * 4), tm)
    h_t, s2 = _stage_b(pf, W0)

    mu = s2[0] / N
    var = s2[1] / N - mu * mu
    rstd = 1.0 / jnp.sqrt(var + EPS)
    scale = gamma * rstd
    shift = beta - mu * scale
    ss = jnp.concatenate([scale, shift]).astype(jnp.float32)

    return _stage_c(h_t, pseg, binfo, ss)


# speedup vs baseline: 1.2015x; 1.2015x over previous
import dataclasses as _dc

import jax
import jax.numpy as jnp
from jax import lax
from jax.experimental import pallas as pl
from jax.experimental.pallas import tpu as pltpu
from jax.experimental.pallas import tpu_sc as plsc

N = 240000
NX, NY, B = 352, 400, 2
NUM_SEG = B * NX * NY
VX = 0.2
VY = 0.2
XOFF = VX / 2 + 0.0
YOFF = VY / 2 + (-40.0)
EPS = 1e-3
C_OUT = 64

NW = 32
WIN = 1600
NWIN = N // WIN
WPT = (NWIN + NW - 1) // NW
GRP = WIN // 16

SPB = 1024
NBINS = 288
NBINS_REAL = NUM_SEG // SPB
NPASS = 9
NEG = -3.0e38

N2 = 128000
CORE_N = (128000, 112000)
CORE_NWIN = (80, 70)

_mesh = plsc.VectorSubcoreMesh(core_axis_name="c", subcore_axis_name="s")
_sc_params = pltpu.CompilerParams(
    needs_layout_passes=False, use_tc_tiling_on_sc=False)

_I16 = lambda: lax.iota(jnp.int32, 16)


def _wid():
    return lax.axis_index("c") * 16 + lax.axis_index("s")


def _sget(vec, lane):
    return jnp.sum(jnp.where(_I16() == lane, vec, 0))


def _stage_a_body(coors_hbm, feats_hbm, seg_hbm, tables_hbm, hist_hbm,
                  cbuf, fbuf, segbuf, ix0, ix1, ix2, ix3, vx, vy, vz,
                  ones_b, zbuf, hcnt, tbl_sh):
    cid = lax.axis_index("c")
    sid = lax.axis_index("s")
    wid = cid * 16 + sid

    @pl.loop(0, 8800, step=16)
    def _(i):
        zbuf[pl.ds(i, 16)] = jnp.zeros((16,), jnp.float32)

    @pl.loop(0, 8, step=1)
    def _(j):
        pltpu.sync_copy(zbuf, tbl_sh.at[pl.ds(sid * 70400 + j * 8800, 8800)])

    @pl.loop(0, WIN, step=16)
    def _(i):
        ones_b[pl.ds(i, 16)] = jnp.ones((16,), jnp.float32)

    @pl.loop(0, NBINS, step=16)
    def _(i):
        hcnt[pl.ds(i, 16)] = jnp.zeros((16,), jnp.int32)

    occ_p, _ = plsc.scan_count(_I16())
    occ_base = jnp.min(occ_p)

    plsc.subcore_barrier()

    @pl.loop(0, WPT)
    def _(k):
        widx = wid + k * NW

        @pl.when(widx < NWIN)
        def _():
            base = widx * WIN
            pltpu.sync_copy(coors_hbm.at[pl.ds(base, WIN)], cbuf)
            pltpu.sync_copy(feats_hbm.at[pl.ds(base, WIN)], fbuf)

            @pl.loop(0, GRP)
            def _(g):
                rows = g * 16 + _I16()
                zc = jnp.zeros((16,), jnp.int32)
                b16 = plsc.load_gather(cbuf, [rows, zc])
                y16 = plsc.load_gather(cbuf, [rows, zc + 2])
                x16 = plsc.load_gather(cbuf, [rows, zc + 3])
                s16 = b16 * (NX * NY) + y16 * NX + x16
                segbuf[pl.ds(g * 16, 16)] = s16
                t4 = s16 * 4
                ix0[pl.ds(g * 16, 16)] = t4
                ix1[pl.ds(g * 16, 16)] = t4 + 1
                ix2[pl.ds(g * 16, 16)] = t4 + 2
                ix3[pl.ds(g * 16, 16)] = t4 + 3
                vx[pl.ds(g * 16, 16)] = plsc.load_gather(fbuf, [rows, zc])
                vy[pl.ds(g * 16, 16)] = plsc.load_gather(fbuf, [rows, zc + 1])
                vz[pl.ds(g * 16, 16)] = plsc.load_gather(fbuf, [rows, zc + 2])

                bin16 = s16 >> 10
                occ, lastm = plsc.scan_count(bin16)
                occ0 = occ - occ_base
                cur = plsc.load_gather(hcnt, [bin16])
                plsc.store_scatter(hcnt, [bin16], cur + occ0 + 1, mask=lastm)

            pltpu.sync_copy(vx, tbl_sh.at[ix0], add=True)
            pltpu.sync_copy(vy, tbl_sh.at[ix1], add=True)
            pltpu.sync_copy(vz, tbl_sh.at[ix2], add=True)
            pltpu.sync_copy(ones_b, tbl_sh.at[ix3], add=True)
            pltpu.sync_copy(segbuf, seg_hbm.at[pl.ds(base, WIN)])

    pltpu.sync_copy(hcnt, hist_hbm.at[pl.ds(wid * NBINS, NBINS)])
    plsc.subcore_barrier()
    pltpu.sync_copy(tbl_sh.at[pl.ds(sid * 70400, 70400)],
                    tables_hbm.at[cid, pl.ds(sid * 70400, 70400)])


def _stage_a(coors, features):
    k = pl.kernel(
        _stage_a_body,
        name="stage_a",
        out_type=(
            jax.ShapeDtypeStruct((N,), jnp.int32),
            jax.ShapeDtypeStruct((2, NUM_SEG * 4), jnp.float32),
            jax.ShapeDtypeStruct((NW * NBINS,), jnp.int32),
        ),
        mesh=_mesh,
        compiler_params=_sc_params,
        scratch_types=[
            pltpu.VMEM((WIN, 4), jnp.int32),
            pltpu.VMEM((WIN, 4), jnp.float32),
            pltpu.VMEM((WIN,), jnp.int32),
            pltpu.VMEM((WIN,), jnp.int32),
            pltpu.VMEM((WIN,), jnp.int32),
            pltpu.VMEM((WIN,), jnp.int32),
            pltpu.VMEM((WIN,), jnp.int32),
            pltpu.VMEM((WIN,), jnp.float32),
            pltpu.VMEM((WIN,), jnp.float32),
            pltpu.VMEM((WIN,), jnp.float32),
            pltpu.VMEM((WIN,), jnp.float32),
            pltpu.VMEM((8800,), jnp.float32),
            pltpu.VMEM((NBINS,), jnp.int32),
            pltpu.VMEM_SHARED((NUM_SEG * 4,), jnp.float32),
        ],
    )
    return k(coors, features)


def _stage_abin_body(seg_hbm, hist_hbm, ppid_hbm, pseg_hbm, binfo_hbm,
                     histbuf, csum, starts, counters,
                     segbuf, posbuf, pidbuf, sp_sh):
    cid = lax.axis_index("c")
    sid = lax.axis_index("s")
    wid = cid * 16 + sid

    pltpu.sync_copy(hist_hbm, histbuf)

    occ_p, _ = plsc.scan_count(_I16())
    occ_base = jnp.min(occ_p)

    @pl.loop(0, NBINS, step=16)
    def _(cb):
        acc = jnp.zeros((16,), jnp.int32)

        def _accf(t, a):
            return a + histbuf[pl.ds(t * NBINS + cb, 16)]

        acc = lax.fori_loop(cid * 16, cid * 16 + 16, _accf, acc)
        csum[pl.ds(cb, 16)] = acc

    def _pref(i, carry):
        v = csum[pl.ds(i * 16, 16)]
        cs = plsc.cumsum(v)
        starts[pl.ds(i * 16, 16)] = carry + cs - v
        return carry + jnp.sum(v)

    lax.fori_loop(0, NBINS // 16, _pref, jnp.int32(0))

    @pl.loop(0, NBINS, step=16)
    def _(cb):
        acc = starts[pl.ds(cb, 16)]

        def _acc(t, a):
            return a + histbuf[pl.ds(t * NBINS + cb, 16)]

        acc = lax.fori_loop(cid * 16, wid, _acc, acc)
        counters[pl.ds(cb, 16)] = acc

    @pl.when(sid == 0)
    def _():
        pltpu.sync_copy(starts, binfo_hbm.at[pl.ds(cid * 2 * NBINS, NBINS)])
        pltpu.sync_copy(csum,
                        binfo_hbm.at[pl.ds(cid * 2 * NBINS + NBINS, NBINS)])

    @pl.loop(0, WPT)
    def _(k):
        widx = wid + k * NW

        @pl.when(widx < NWIN)
        def _():
            base = widx * WIN
            pltpu.sync_copy(seg_hbm.at[pl.ds(base, WIN)], segbuf)

            @pl.loop(0, GRP)
            def _(g):
                s16 = segbuf[pl.ds(g * 16, 16)]
                bin16 = s16 >> 10
                occ, lastm = plsc.scan_count(bin16)
                occ0 = occ - occ_base
                cur = plsc.load_gather(counters, [bin16])
                plsc.store_scatter(counters, [bin16], cur + occ0 + 1,
                                   mask=lastm)
                posbuf[pl.ds(g * 16, 16)] = cur + occ0
                pidbuf[pl.ds(g * 16, 16)] = base + g * 16 + _I16()

            pltpu.sync_copy(pidbuf, sp_sh.at[posbuf])

            @pl.loop(0, GRP)
            def _(g):
                posbuf[pl.ds(g * 16, 16)] = posbuf[pl.ds(g * 16, 16)] + N2

            pltpu.sync_copy(segbuf, sp_sh.at[posbuf])

    plsc.subcore_barrier()
    pltpu.sync_copy(sp_sh.at[pl.ds(sid * 8000, 8000)],
                    ppid_hbm.at[cid, pl.ds(sid * 8000, 8000)])
    pltpu.sync_copy(sp_sh.at[pl.ds(N2 + sid * 8000, 8000)],
                    pseg_hbm.at[cid, pl.ds(sid * 8000, 8000)])


def _stage_abin(seg, hist):
    k = pl.kernel(
        _stage_abin_body,
        name="stage_abin",
        out_type=(
            jax.ShapeDtypeStruct((2, N2), jnp.int32),
            jax.ShapeDtypeStruct((2, N2), jnp.int32),
            jax.ShapeDtypeStruct((4 * NBINS,), jnp.int32),
        ),
        mesh=_mesh,
        compiler_params=_sc_params,
        scratch_types=[
            pltpu.VMEM((NW * NBINS,), jnp.int32),
            pltpu.VMEM((NBINS,), jnp.int32),
            pltpu.VMEM((NBINS,), jnp.int32),
            pltpu.VMEM((NBINS,), jnp.int32),
            pltpu.VMEM((WIN,), jnp.int32),
            pltpu.VMEM((WIN,), jnp.int32),
            pltpu.VMEM((WIN,), jnp.int32),
            pltpu.VMEM_SHARED((2 * N2,), jnp.int32),
        ],
    )
    return k(seg, hist)


MCH = 6400
NMCH = NUM_SEG * 4 // MCH


def _stage_amerge_body(t0_hbm, t1_hbm, tm_hbm, b0, b1, ob):
    wid = _wid()

    @pl.loop(0, (NMCH + NW - 1) // NW)
    def _(k):
        ci = wid + k * NW

        @pl.when(ci < NMCH)
        def _():
            base = ci * MCH
            pltpu.sync_copy(t0_hbm.at[pl.ds(base, MCH)], b0)
            pltpu.sync_copy(t1_hbm.at[pl.ds(base, MCH)], b1)

            @pl.loop(0, MCH // 64)
            def _(g):
                rows4 = (g * 16 + _I16()) * 4
                sx = (plsc.load_gather(b0, [rows4])
                      + plsc.load_gather(b1, [rows4]))
                sy = (plsc.load_gather(b0, [rows4 + 1])
                      + plsc.load_gather(b1, [rows4 + 1]))
                sz = (plsc.load_gather(b0, [rows4 + 2])
                      + plsc.load_gather(b1, [rows4 + 2]))
                cnt = (plsc.load_gather(b0, [rows4 + 3])
                       + plsc.load_gather(b1, [rows4 + 3]))
                inv = 1.0 / jnp.maximum(cnt, 1.0)
                plsc.store_scatter(ob, [rows4], sx * inv)
                plsc.store_scatter(ob, [rows4 + 1], sy * inv)
                plsc.store_scatter(ob, [rows4 + 2], sz * inv)
                plsc.store_scatter(ob, [rows4 + 3], cnt)

            pltpu.sync_copy(ob, tm_hbm.at[pl.ds(base, MCH)])


def _stage_amerge(tables):
    k = pl.kernel(
        _stage_amerge_body,
        name="stage_amerge",
        out_type=jax.ShapeDtypeStruct((NUM_SEG * 4,), jnp.float32),
        mesh=_mesh,
        compiler_params=_sc_params,
        scratch_types=[
            pltpu.VMEM((MCH,), jnp.float32),
            pltpu.VMEM((MCH,), jnp.float32),
            pltpu.VMEM((MCH,), jnp.float32),
        ],
    )
    return k(tables[0], tables[1])


def _stage_a2_body(ppid_hbm, pseg_hbm, featf_hbm, tm_hbm, pf_hbm,
                   pidb, segb, jf0, jf1, jf2, jf3, jm0, jm1, jm2,
                   r0, r1, r2, r3, r4, r5, r6, r7, r8):
    wid = _wid()
    out_rows = (r0, r1, r2, r3, r4, r5, r6, r7, r8)

    @pl.loop(0, WPT)
    def _(k):
        widx = wid + k * NW

        @pl.when(widx < NWIN)
        def _():
            base = widx * WIN
            src = (widx >= CORE_NWIN[0]).astype(jnp.int32)
            inbase = base - src * N2
            pltpu.sync_copy(ppid_hbm.at[src, pl.ds(inbase, WIN)], pidb)
            pltpu.sync_copy(pseg_hbm.at[src, pl.ds(inbase, WIN)], segb)

            @pl.loop(0, GRP)
            def _(g):
                sl = pl.ds(g * 16, 16)
                pid = pidb[sl]
                pid = jnp.minimum(jnp.maximum(pid, 0), N - 1)
                s16 = segb[sl]
                s16 = jnp.minimum(jnp.maximum(s16, 0), NUM_SEG - 1)
                pidb[sl] = pid * 4
                segb[sl] = s16 * 4
                xv = jnp.remainder(s16, NX)
                yv = jnp.remainder(s16 // NX, NY)
                r7[sl] = yv.astype(jnp.float32)
                r8[sl] = xv.astype(jnp.float32)

            pltpu.sync_copy(featf_hbm.at[pidb], jf0)
            pltpu.sync_copy(tm_hbm.at[segb], jm0)

            @pl.loop(0, GRP)
            def _(g):
                sl = pl.ds(g * 16, 16)
                pidb[sl] = pidb[sl] + 1
                segb[sl] = segb[sl] + 1

            pltpu.sync_copy(featf_hbm.at[pidb], jf1)
            pltpu.sync_copy(tm_hbm.at[segb], jm1)

            @pl.loop(0, GRP)
            def _(g):
                sl = pl.ds(g * 16, 16)
                pidb[sl] = pidb[sl] + 1
                segb[sl] = segb[sl] + 1

            pltpu.sync_copy(featf_hbm.at[pidb], jf2)
            pltpu.sync_copy(tm_hbm.at[segb], jm2)

            @pl.loop(0, GRP)
            def _(g):
                sl = pl.ds(g * 16, 16)
                pidb[sl] = pidb[sl] + 1

            pltpu.sync_copy(featf_hbm.at[pidb], jf3)

            @pl.loop(0, GRP)
            def _(g):
                sl = pl.ds(g * 16, 16)
                r0[sl] = jf0[sl]
                r1[sl] = jf1[sl]
                r2[sl] = jf2[sl]
                r3[sl] = jf3[sl]
                r4[sl] = jf0[sl] - jm0[sl]
                r5[sl] = jf1[sl] - jm1[sl]
                r6[sl] = jf2[sl] - jm2[sl]
                fy = jf1[sl] - (r7[sl] * VY + YOFF)
                fx = jf0[sl] - (r8[sl] * VX + XOFF)
                r7[sl] = fx
                r8[sl] = fy

            for c in range(9):
                pltpu.sync_copy(out_rows[c], pf_hbm.at[c, pl.ds(base, WIN)])


def _stage_a2(ppid, pseg, featf, tm):
    k = pl.kernel(
        _stage_a2_body,
        name="stage_a2",
        out_type=jax.ShapeDtypeStruct((9, N), jnp.float32),
        mesh=_mesh,
        compiler_params=_sc_params,
        scratch_types=(
            [pltpu.VMEM((WIN,), jnp.int32)] * 2
            + [pltpu.VMEM((WIN,), jnp.float32)] * 16
        ),
    )
    return k(ppid, pseg, featf, tm)


BLK = 3200
NBLK = N // BLK


def _stage_b_kernel(pf_ref, w_ref, h_ref, s_ref):
    i = pl.program_id(0)
    f = pf_ref[...]
    w = w_ref[...]
    h = jnp.dot(w.T, f, preferred_element_type=jnp.float32)
    h_ref[...] = h

    @pl.when(i == 0)
    def _():
        s_ref[...] = jnp.zeros_like(s_ref)

    s_ref[0:1, :] += jnp.sum(h, axis=1, keepdims=True).T
    s_ref[1:2, :] += jnp.sum(h * h, axis=1, keepdims=True).T


def _stage_b(pf, w9):
    return pl.pallas_call(
        _stage_b_kernel,
        grid=(NBLK,),
        in_specs=[
            pl.BlockSpec((9, BLK), lambda i: (0, i)),
            pl.BlockSpec((9, C_OUT), lambda i: (0, 0)),
        ],
        out_specs=[
            pl.BlockSpec((C_OUT, BLK), lambda i: (0, i)),
            pl.BlockSpec((2, C_OUT), lambda i: (0, 0)),
        ],
        out_shape=[
            jax.ShapeDtypeStruct((C_OUT, N), jnp.float32),
            jax.ShapeDtypeStruct((2, C_OUT), jnp.float32),
        ],
    )(pf, w9)


CH = 512


def _stage_c_body(ht_hbm, pseg_hbm, binfo_hbm, ss_hbm, out_hbm,
                  infob, ssb, tbl, sbuf, vbuf):
    wid = _wid()

    pltpu.sync_copy(binfo_hbm, infob)
    pltpu.sync_copy(ss_hbm, ssb)

    occ_p, _ = plsc.scan_count(_I16())
    occ_base = jnp.min(occ_p)

    @pl.loop(0, NPASS)
    def _(p):
        b = p * NW + wid

        @pl.when(b < NBINS_REAL)
        def _():
            bq = b >> 4
            br = b - bq * 16
            segbase = b * SPB

            negv = jnp.full((16,), NEG, jnp.float32)

            @pl.loop(0, SPB, step=4)
            def _(i):
                for u in range(4):
                    for j in range(4):
                        tbl[i + u, pl.ds(j * 16, 16)] = negv

            for c in range(2):
                start = _sget(infob[pl.ds(c * 2 * NBINS + bq * 16, 16)], br)
                blen = _sget(
                    infob[pl.ds(c * 2 * NBINS + NBINS + bq * 16, 16)], br)
                astart = pl.multiple_of((start >> 4) << 4, 16)
                total = start + blen - astart
                nch = (total + CH - 1) // CH

                @pl.loop(0, nch)
                def _(ci):
                    pos = pl.multiple_of(astart + ci * CH, 16)
                    pltpu.sync_copy(pseg_hbm.at[c, pl.ds(pos, CH)], sbuf)
                    pltpu.sync_copy(ht_hbm.at[:, pl.ds(c * N2 + pos, CH)],
                                    vbuf)
                    ngrp = jnp.minimum(total - ci * CH + 15, CH) >> 4

                    @pl.loop(0, ngrp)
                    def _(g):
                        rows = g * 16 + _I16()
                        gidx = ci * CH + rows
                        valid = ((astart + gidx >= start)
                                 & (astart + gidx < start + blen))
                        s16 = sbuf[pl.ds(g * 16, 16)]
                        soff = s16 - segbase
                        soff = jnp.minimum(jnp.maximum(soff, 0), SPB - 1)
                        occ, _lm = plsc.scan_count(soff, mask=valid)
                        occ0 = jnp.where(valid, occ - occ_base, 0)
                        mo = jnp.max(occ0)
                        zc = jnp.zeros((16,), jnp.int32)

                        @pl.loop(0, mo + 1)
                        def _(r):
                            mr = valid & (occ0 == r)
                            for ch in range(C_OUT):
                                v = vbuf[ch, pl.ds(g * 16, 16)]
                                t = plsc.load_gather(tbl, [soff, zc + ch],
                                                     mask=mr)
                                plsc.store_scatter(tbl, [soff, zc + ch],
                                                   jnp.maximum(t, v),
                                                   mask=mr)

            scs = [ssb[pl.ds(16 * j, 16)] for j in range(4)]
            shs = [ssb[pl.ds(64 + 16 * j, 16)] for j in range(4)]

            @pl.loop(0, SPB, step=4)
            def _(r):
                for u in range(4):
                    for j in range(4):
                        sl = pl.ds(16 * j, 16)
                        tbl[r + u, sl] = jnp.maximum(
                            tbl[r + u, sl] * scs[j] + shs[j], 0.0)

            pltpu.sync_copy(tbl, out_hbm.at[pl.ds(b * SPB, SPB), :])


def _stage_c(ht, pseg, binfo, ss):
    k = pl.kernel(
        _stage_c_body,
        name="stage_c",
        out_type=jax.ShapeDtypeStruct((NUM_SEG, C_OUT), jnp.float32),
        mesh=_mesh,
        compiler_params=_sc_params,
        scratch_types=[
            pltpu.VMEM((4 * NBINS,), jnp.int32),
            pltpu.VMEM((128,), jnp.float32),
            pltpu.VMEM((SPB, C_OUT), jnp.float32),
            pltpu.VMEM((CH,), jnp.int32),
            pltpu.VMEM((C_OUT, CH), jnp.float32),
        ],
    )
    return k(ht, pseg, binfo, ss)


def kernel(features, coors, W0, gamma, beta):
    seg, tables, hist = _stage_a(coors, features)
    ppid, pseg, binfo = _stage_abin(seg, hist)
    tm = _stage_amerge(tables)
    pf = _stage_a2(ppid, pseg, features.reshape(N * 4), tm)
    h_t, s2 = _stage_b(pf, W0)

    mu = s2[0] / N
    var = s2[1] / N - mu * mu
    rstd = 1.0 / jnp.sqrt(var + EPS)
    scale = gamma * rstd
    shift = beta - mu * scale
    ss = jnp.concatenate([scale, shift]).astype(jnp.float32)

    return _stage_c(h_t, pseg, binfo, ss)
